# Initial kernel scaffold; baseline (speedup 1.0000x reference)
import jax
import jax.numpy as jnp
from jax.experimental import pallas as pl


def kernel(x, t, edge_index, edge_attr, time_w, time_b, W1_0, W2_0, W1_1, W2_1, fc1_w, fc1_b, fc2_w, fc2_b):
    raise NotImplementedError("write your pallas kernel here")


# speedup vs baseline: 4.8488x; 4.8488x over previous
import functools

import jax
import jax.numpy as jnp
from jax import lax
from jax.experimental import pallas as pl
from jax.experimental.pallas import tpu as pltpu
from jax.experimental.pallas import tpu_sc as plsc

N_NODES = 10000
N_EDGES = 320000
D_FEAT = 128
D_EDGE = 20
D_HID = 256
D_OUT = 2

NC = 2
NS = 16
NW = NC * NS
EPW = N_EDGES // NW
CHUNK = 128
NCHUNK = EPW // CHUNK
TAIL = EPW - NCHUNK * CHUNK
LANES = 16
STRIPE = 640
WCHUNK = 80

_SC_MESH = plsc.VectorSubcoreMesh(core_axis_name="c", subcore_axis_name="s")
_SC_PARAMS = pltpu.CompilerParams(needs_layout_passes=False)


def _dt_body(t_hbm, src_hbm, dst_hbm, dt_hbm, t_v, src_v, dst_v, dt_v):
    c = lax.axis_index("c")
    s = lax.axis_index("s")
    wid = s * NC + c
    base = wid * EPW
    pltpu.sync_copy(t_hbm, t_v)
    pltpu.sync_copy(src_hbm.at[pl.ds(base, EPW)], src_v)
    pltpu.sync_copy(dst_hbm.at[pl.ds(base, EPW)], dst_v)

    def body(i, carry):
        sl = pl.ds(i * LANES, LANES)
        ts = plsc.load_gather(t_v, [src_v[sl]])
        td = plsc.load_gather(t_v, [dst_v[sl]])
        dt_v[sl] = td - ts
        return carry

    lax.fori_loop(0, EPW // LANES, body, 0)
    pltpu.sync_copy(dt_v, dt_hbm.at[pl.ds(base, EPW)])


_dt_kernel = pl.kernel(
    _dt_body,
    mesh=_SC_MESH,
    compiler_params=_SC_PARAMS,
    out_type=jax.ShapeDtypeStruct((N_EDGES,), jnp.float32),
    scratch_types=[
        pltpu.VMEM((N_NODES,), jnp.float32),
        pltpu.VMEM((EPW,), jnp.int32),
        pltpu.VMEM((EPW,), jnp.int32),
        pltpu.VMEM((EPW,), jnp.float32),
    ],
)


_BE = 2000


def _ec_body(dt_ref, ea_ref, w_ref, b_ref, wb0_ref, wc0_ref, wb1_ref, wc1_ref,
             ec0_ref, ec1_ref):
    te = jnp.cos(dt_ref[...] * w_ref[...] + b_ref[...])
    ea = ea_ref[...]
    ec0_ref[...] = (
        jnp.dot(ea, wb0_ref[...], preferred_element_type=jnp.float32)
        + jnp.dot(te, wc0_ref[...], preferred_element_type=jnp.float32))
    ec1_ref[...] = (
        jnp.dot(ea, wb1_ref[...], preferred_element_type=jnp.float32)
        + jnp.dot(te, wc1_ref[...], preferred_element_type=jnp.float32))


_ec_kernel = pl.pallas_call(
    _ec_body,
    grid=(N_EDGES // _BE,),
    in_specs=[
        pl.BlockSpec((_BE, 1), lambda i: (i, 0)),
        pl.BlockSpec((_BE, D_EDGE), lambda i: (i, 0)),
        pl.BlockSpec((1, D_FEAT), lambda i: (0, 0)),
        pl.BlockSpec((1, D_FEAT), lambda i: (0, 0)),
        pl.BlockSpec((D_EDGE, D_FEAT), lambda i: (0, 0)),
        pl.BlockSpec((D_FEAT, D_FEAT), lambda i: (0, 0)),
        pl.BlockSpec((D_EDGE, D_FEAT), lambda i: (0, 0)),
        pl.BlockSpec((D_FEAT, D_FEAT), lambda i: (0, 0)),
    ],
    out_specs=[
        pl.BlockSpec((_BE, D_FEAT), lambda i: (i, 0)),
        pl.BlockSpec((_BE, D_FEAT), lambda i: (i, 0)),
    ],
    out_shape=[
        jax.ShapeDtypeStruct((N_EDGES, D_FEAT), jnp.float32),
        jax.ShapeDtypeStruct((N_EDGES, D_FEAT), jnp.float32),
    ],
)


_BN = 1000


def _g_body(h_ref, w_ref, g_ref):
    g_ref[...] = jnp.dot(h_ref[...], w_ref[...],
                         preferred_element_type=jnp.float32)


_g_kernel = pl.pallas_call(
    _g_body,
    grid=(N_NODES // _BN,),
    in_specs=[
        pl.BlockSpec((_BN, D_FEAT), lambda i: (i, 0)),
        pl.BlockSpec((D_FEAT, D_FEAT), lambda i: (0, 0)),
    ],
    out_specs=pl.BlockSpec((_BN, D_FEAT), lambda i: (i, 0)),
    out_shape=jax.ShapeDtypeStruct((N_NODES, D_FEAT), jnp.float32),
)


def _edge_body(g_hbm, ec_hbm, src_hbm, dst_hbm, out_hbm,
               src_c, dst_c, src_t, dst_t, gbuf, ecbuf, tbuf, tebuf,
               agg_sh, sem):
    c = lax.axis_index("c")
    s = lax.axis_index("s")
    wid = s * NC + c
    base = wid * EPW
    nwb = jnp.where(s < NS - 1, STRIPE // WCHUNK, (N_NODES - (NS - 1) * STRIPE) // WCHUNK)

    def zrow(i, carry):
        for j in range(D_FEAT // LANES):
            gbuf[i, pl.ds(j * LANES, LANES)] = jnp.zeros((LANES,), jnp.float32)
        return carry

    lax.fori_loop(0, CHUNK, zrow, 0)

    def zstripe(q, carry):
        pltpu.sync_copy(gbuf.at[pl.ds(0, WCHUNK)],
                        agg_sh.at[pl.ds(s * STRIPE + q * WCHUNK, WCHUNK)])
        return carry

    lax.fori_loop(0, nwb, zstripe, 0)
    plsc.subcore_barrier()

    def chunk_body(ci, carry):
        e0 = base + ci * CHUNK
        pltpu.sync_copy(src_hbm.at[pl.ds(e0, CHUNK)], src_c)
        pltpu.sync_copy(dst_hbm.at[pl.ds(e0, CHUNK)], dst_c)
        pltpu.async_copy(g_hbm.at[src_c], gbuf, sem).wait()
        pltpu.sync_copy(ec_hbm.at[pl.ds(e0, CHUNK)], ecbuf)

        def row(i, rcarry):
            for j in range(D_FEAT // LANES):
                sl = pl.ds(j * LANES, LANES)
                gbuf[i, sl] = jnp.maximum(gbuf[i, sl] + ecbuf[i, sl], 0.0)
            return rcarry

        lax.fori_loop(0, CHUNK, row, 0)
        pltpu.sync_copy(gbuf, agg_sh.at[dst_c], add=True)
        return carry

    lax.fori_loop(0, NCHUNK, chunk_body, 0)

    e0 = base + NCHUNK * CHUNK
    pltpu.sync_copy(src_hbm.at[pl.ds(e0, TAIL)], src_t)
    pltpu.sync_copy(dst_hbm.at[pl.ds(e0, TAIL)], dst_t)
    pltpu.async_copy(g_hbm.at[src_t], tbuf, sem).wait()
    pltpu.sync_copy(ec_hbm.at[pl.ds(e0, TAIL)], tebuf)

    def trow(i, rcarry):
        for j in range(D_FEAT // LANES):
            sl = pl.ds(j * LANES, LANES)
            tbuf[i, sl] = jnp.maximum(tbuf[i, sl] + tebuf[i, sl], 0.0)
        return rcarry

    lax.fori_loop(0, TAIL, trow, 0)
    pltpu.sync_copy(tbuf, agg_sh.at[dst_t], add=True)
    plsc.subcore_barrier()

    def wstripe(q, carry):
        r0 = s * STRIPE + q * WCHUNK
        pltpu.sync_copy(agg_sh.at[pl.ds(r0, WCHUNK)], gbuf.at[pl.ds(0, WCHUNK)])
        pltpu.sync_copy(gbuf.at[pl.ds(0, WCHUNK)], out_hbm.at[c, pl.ds(r0, WCHUNK)])
        return carry

    lax.fori_loop(0, nwb, wstripe, 0)


_edge_kernel = pl.kernel(
    _edge_body,
    mesh=_SC_MESH,
    compiler_params=_SC_PARAMS,
    out_type=jax.ShapeDtypeStruct((NC, N_NODES, D_FEAT), jnp.float32),
    scratch_types=[
        pltpu.VMEM((CHUNK,), jnp.int32),
        pltpu.VMEM((CHUNK,), jnp.int32),
        pltpu.VMEM((TAIL,), jnp.int32),
        pltpu.VMEM((TAIL,), jnp.int32),
        pltpu.VMEM((CHUNK, D_FEAT), jnp.float32),
        pltpu.VMEM((CHUNK, D_FEAT), jnp.float32),
        pltpu.VMEM((TAIL, D_FEAT), jnp.float32),
        pltpu.VMEM((TAIL, D_FEAT), jnp.float32),
        pltpu.VMEM_SHARED((N_NODES, D_FEAT), jnp.float32),
        pltpu.SemaphoreType.DMA,
    ],
)


def _hup_body(h_ref, aa_ref, ab_ref, w2t_ref, w2b_ref, w1a_ref, hn_ref, gn_ref):
    agg = aa_ref[...] + ab_ref[...]
    hn = jnp.maximum(
        jnp.dot(h_ref[...], w2t_ref[...], preferred_element_type=jnp.float32)
        + jnp.dot(agg, w2b_ref[...], preferred_element_type=jnp.float32), 0.0)
    hn_ref[...] = hn
    gn_ref[...] = jnp.dot(hn, w1a_ref[...], preferred_element_type=jnp.float32)


_hup_kernel = pl.pallas_call(
    _hup_body,
    grid=(N_NODES // _BN,),
    in_specs=[
        pl.BlockSpec((_BN, D_FEAT), lambda i: (i, 0)),
        pl.BlockSpec((_BN, D_FEAT), lambda i: (i, 0)),
        pl.BlockSpec((_BN, D_FEAT), lambda i: (i, 0)),
        pl.BlockSpec((D_FEAT, D_FEAT), lambda i: (0, 0)),
        pl.BlockSpec((D_FEAT, D_FEAT), lambda i: (0, 0)),
        pl.BlockSpec((D_FEAT, D_FEAT), lambda i: (0, 0)),
    ],
    out_specs=[
        pl.BlockSpec((_BN, D_FEAT), lambda i: (i, 0)),
        pl.BlockSpec((_BN, D_FEAT), lambda i: (i, 0)),
    ],
    out_shape=[
        jax.ShapeDtypeStruct((N_NODES, D_FEAT), jnp.float32),
        jax.ShapeDtypeStruct((N_NODES, D_FEAT), jnp.float32),
    ],
)


def _dec_body(h_ref, aa_ref, ab_ref, w2t_ref, w2b_ref,
              fc1w_ref, fc1b_ref, fc2w_ref, fc2b_ref, out_ref):
    agg = aa_ref[...] + ab_ref[...]
    h2 = jnp.maximum(
        jnp.dot(h_ref[...], w2t_ref[...], preferred_element_type=jnp.float32)
        + jnp.dot(agg, w2b_ref[...], preferred_element_type=jnp.float32), 0.0)
    emb = jnp.dot(h2, fc1w_ref[...],
                  preferred_element_type=jnp.float32) + fc1b_ref[...]
    r = jnp.maximum(emb, 0.0)
    o = jnp.dot(r, fc2w_ref[...],
                preferred_element_type=jnp.float32) + fc2b_ref[...]
    soft = jnp.exp(o - jnp.max(o))
    out_ref[...] = soft / jnp.sum(soft)


_dec_kernel = pl.pallas_call(
    _dec_body,
    out_shape=jax.ShapeDtypeStruct((N_NODES, D_OUT), jnp.float32),
)


def kernel(x, t, edge_index, edge_attr, time_w, time_b,
           W1_0, W2_0, W1_1, W2_1, fc1_w, fc1_b, fc2_w, fc2_b):
    src = edge_index[0].astype(jnp.int32)
    dst = edge_index[1].astype(jnp.int32)

    dt = _dt_kernel(t, src, dst)

    ec0, ec1 = _ec_kernel(
        dt.reshape(N_EDGES, 1), edge_attr,
        time_w.reshape(1, D_FEAT), time_b.reshape(1, D_FEAT),
        W1_0[D_FEAT:D_FEAT + D_EDGE], W1_0[D_FEAT + D_EDGE:],
        W1_1[D_FEAT:D_FEAT + D_EDGE], W1_1[D_FEAT + D_EDGE:])

    g0 = _g_kernel(x, W1_0[:D_FEAT])

    p0 = _edge_kernel(g0, ec0, src, dst)
    h1, g1 = _hup_kernel(x, p0[0], p0[1],
                         W2_0[:D_FEAT], W2_0[D_FEAT:], W1_1[:D_FEAT])

    p1 = _edge_kernel(g1, ec1, src, dst)
    out = _dec_kernel(h1, p1[0], p1[1], W2_1[:D_FEAT], W2_1[D_FEAT:],
                      fc1_w, fc1_b.reshape(1, D_HID),
                      fc2_w, fc2_b.reshape(1, D_OUT))
    return out


# speedup vs baseline: 6.6048x; 1.3622x over previous
import functools

import jax
import jax.numpy as jnp
from jax import lax
from jax.experimental import pallas as pl
from jax.experimental.pallas import tpu as pltpu
from jax.experimental.pallas import tpu_sc as plsc

N_NODES = 10000
N_EDGES = 320000
D_FEAT = 128
D_EDGE = 20
D_HID = 256
D_OUT = 2

NC = 2
NS = 16
NW = NC * NS
EPW = N_EDGES // NW
CHUNK = 40
NCHUNK = EPW // CHUNK
TAIL = EPW - NCHUNK * CHUNK
LANES = 16
STRIPE = 640
WCHUNK = 40

_SC_MESH = plsc.VectorSubcoreMesh(core_axis_name="c", subcore_axis_name="s")
_SC_PARAMS = pltpu.CompilerParams(needs_layout_passes=False)


def _dt_body(t_hbm, src_hbm, dst_hbm, dt_hbm, t_v, src_v, dst_v, dt_v):
    c = lax.axis_index("c")
    s = lax.axis_index("s")
    wid = s * NC + c
    base = wid * EPW
    pltpu.sync_copy(t_hbm, t_v)
    pltpu.sync_copy(src_hbm.at[pl.ds(base, EPW)], src_v)
    pltpu.sync_copy(dst_hbm.at[pl.ds(base, EPW)], dst_v)

    def body(i, carry):
        sl = pl.ds(i * LANES, LANES)
        ts = plsc.load_gather(t_v, [src_v[sl]])
        td = plsc.load_gather(t_v, [dst_v[sl]])
        dt_v[sl] = td - ts
        return carry

    lax.fori_loop(0, EPW // LANES, body, 0)
    pltpu.sync_copy(dt_v, dt_hbm.at[pl.ds(base, EPW)])


_dt_kernel = pl.kernel(
    _dt_body,
    mesh=_SC_MESH,
    compiler_params=_SC_PARAMS,
    out_type=jax.ShapeDtypeStruct((N_EDGES,), jnp.float32),
    scratch_types=[
        pltpu.VMEM((N_NODES,), jnp.float32),
        pltpu.VMEM((EPW,), jnp.int32),
        pltpu.VMEM((EPW,), jnp.int32),
        pltpu.VMEM((EPW,), jnp.float32),
    ],
)


_BE = 2000


def _ec_body(dt_ref, ea_ref, w_ref, b_ref, wb0_ref, wc0_ref, wb1_ref, wc1_ref,
             ec0_ref, ec1_ref):
    te = jnp.cos(dt_ref[...] * w_ref[...] + b_ref[...])
    ea = ea_ref[...]
    ec0_ref[...] = (
        jnp.dot(ea, wb0_ref[...], preferred_element_type=jnp.float32)
        + jnp.dot(te, wc0_ref[...], preferred_element_type=jnp.float32))
    ec1_ref[...] = (
        jnp.dot(ea, wb1_ref[...], preferred_element_type=jnp.float32)
        + jnp.dot(te, wc1_ref[...], preferred_element_type=jnp.float32))


_ec_kernel = pl.pallas_call(
    _ec_body,
    grid=(N_EDGES // _BE,),
    in_specs=[
        pl.BlockSpec((_BE, 1), lambda i: (i, 0)),
        pl.BlockSpec((_BE, D_EDGE), lambda i: (i, 0)),
        pl.BlockSpec((1, D_FEAT), lambda i: (0, 0)),
        pl.BlockSpec((1, D_FEAT), lambda i: (0, 0)),
        pl.BlockSpec((D_EDGE, D_FEAT), lambda i: (0, 0)),
        pl.BlockSpec((D_FEAT, D_FEAT), lambda i: (0, 0)),
        pl.BlockSpec((D_EDGE, D_FEAT), lambda i: (0, 0)),
        pl.BlockSpec((D_FEAT, D_FEAT), lambda i: (0, 0)),
    ],
    out_specs=[
        pl.BlockSpec((_BE, D_FEAT), lambda i: (i, 0)),
        pl.BlockSpec((_BE, D_FEAT), lambda i: (i, 0)),
    ],
    out_shape=[
        jax.ShapeDtypeStruct((N_EDGES, D_FEAT), jnp.float32),
        jax.ShapeDtypeStruct((N_EDGES, D_FEAT), jnp.float32),
    ],
)


_BN = 1000


def _g_body(h_ref, w_ref, g_ref):
    g_ref[...] = jnp.dot(h_ref[...], w_ref[...],
                         preferred_element_type=jnp.float32)


_g_kernel = pl.pallas_call(
    _g_body,
    grid=(N_NODES // _BN,),
    in_specs=[
        pl.BlockSpec((_BN, D_FEAT), lambda i: (i, 0)),
        pl.BlockSpec((D_FEAT, D_FEAT), lambda i: (0, 0)),
    ],
    out_specs=pl.BlockSpec((_BN, D_FEAT), lambda i: (i, 0)),
    out_shape=jax.ShapeDtypeStruct((N_NODES, D_FEAT), jnp.float32),
)


def _edge_body(g_hbm, ec_hbm, src_hbm, dst_hbm, out_hbm,
               src_all, dst_c2, gb, eb, ob,
               agg_sh, gsems, esems, dsems, ssems):
    c = lax.axis_index("c")
    s = lax.axis_index("s")
    wid = s * NC + c
    base = wid * EPW
    nwb = jnp.where(s < NS - 1, STRIPE // WCHUNK, (N_NODES - (NS - 1) * STRIPE) // WCHUNK)

    def zrow(i, carry):
        for j in range(D_FEAT // LANES):
            ob[0, i, pl.ds(j * LANES, LANES)] = jnp.zeros((LANES,), jnp.float32)
        return carry

    lax.fori_loop(0, WCHUNK, zrow, 0)

    def zstripe(q, carry):
        pltpu.sync_copy(ob.at[0, pl.ds(0, WCHUNK)],
                        agg_sh.at[pl.ds(s * STRIPE + q * WCHUNK, WCHUNK)])
        return carry

    lax.fori_loop(0, nwb, zstripe, 0)
    pltpu.sync_copy(src_hbm.at[pl.ds(base, EPW)], src_all)
    plsc.subcore_barrier()

    def prefetch(ci, b, d):
        e0 = base + ci * CHUNK
        pltpu.async_copy(dst_hbm.at[pl.ds(e0, CHUNK)], dst_c2.at[d], dsems[d])
        pltpu.async_copy(g_hbm.at[src_all.at[pl.ds(ci * CHUNK, CHUNK)]],
                         gb.at[b], gsems[b])
        pltpu.async_copy(ec_hbm.at[pl.ds(e0, CHUNK)], eb.at[b], esems[b])

    def process(ci, b, d, maybe_first, do_prefetch):
        pltpu.make_async_copy(g_hbm.at[src_all.at[pl.ds(0, CHUNK)]],
                              gb.at[b], gsems[b]).wait()
        pltpu.make_async_copy(ec_hbm.at[pl.ds(base, CHUNK)],
                              eb.at[b], esems[b]).wait()

        def swait():
            pltpu.make_async_copy(ob.at[b], agg_sh.at[dst_c2.at[d]],
                                  ssems[b]).wait()

        if maybe_first is None:
            swait()
        else:
            pl.when(maybe_first > 0)(swait)

        def row(i, rcarry):
            for j in range(D_FEAT // LANES):
                sl = pl.ds(j * LANES, LANES)
                ob[b, i, sl] = jnp.maximum(gb[b, i, sl] + eb[b, i, sl], 0.0)
            return rcarry

        lax.fori_loop(0, CHUNK, row, 0)
        pltpu.make_async_copy(dst_hbm.at[pl.ds(base, CHUNK)],
                              dst_c2.at[d], dsems[d]).wait()
        pltpu.async_copy(ob.at[b], agg_sh.at[dst_c2.at[d]], ssems[b], add=True)
        if do_prefetch:
            prefetch(ci + 2, b, (d + 2) % 4)

    prefetch(0, 0, 0)
    prefetch(1, 1, 1)

    NQ = (NCHUNK - 2) // 4

    def quad_body(q, carry):
        for u in range(4):
            process(4 * q + u, u & 1, u, q if u < 2 else None, True)
        return carry

    lax.fori_loop(0, NQ, quad_body, 0)
    process(NCHUNK - 2, 0, (NCHUNK - 2) % 4, None, False)
    process(NCHUNK - 1, 1, (NCHUNK - 1) % 4, None, False)

    for b in range(2):
        pltpu.make_async_copy(ob.at[b], agg_sh.at[dst_c2.at[b]],
                              ssems[b]).wait()
    plsc.subcore_barrier()

    def wstripe(q, carry):
        r0 = s * STRIPE + q * WCHUNK
        pltpu.sync_copy(agg_sh.at[pl.ds(r0, WCHUNK)], ob.at[0, pl.ds(0, WCHUNK)])
        pltpu.sync_copy(ob.at[0, pl.ds(0, WCHUNK)], out_hbm.at[c, pl.ds(r0, WCHUNK)])
        return carry

    lax.fori_loop(0, nwb, wstripe, 0)


_edge_kernel = pl.kernel(
    _edge_body,
    mesh=_SC_MESH,
    compiler_params=_SC_PARAMS,
    out_type=jax.ShapeDtypeStruct((NC, N_NODES, D_FEAT), jnp.float32),
    scratch_types=[
        pltpu.VMEM((EPW,), jnp.int32),
        pltpu.VMEM((4, CHUNK), jnp.int32),
        pltpu.VMEM((2, CHUNK, D_FEAT), jnp.float32),
        pltpu.VMEM((2, CHUNK, D_FEAT), jnp.float32),
        pltpu.VMEM((2, CHUNK, D_FEAT), jnp.float32),
        pltpu.VMEM_SHARED((N_NODES, D_FEAT), jnp.float32),
        [pltpu.SemaphoreType.DMA, pltpu.SemaphoreType.DMA],
        [pltpu.SemaphoreType.DMA, pltpu.SemaphoreType.DMA],
        [pltpu.SemaphoreType.DMA, pltpu.SemaphoreType.DMA,
         pltpu.SemaphoreType.DMA, pltpu.SemaphoreType.DMA],
        [pltpu.SemaphoreType.DMA, pltpu.SemaphoreType.DMA],
    ],
)


def _hup_body(h_ref, aa_ref, ab_ref, w2t_ref, w2b_ref, w1a_ref, hn_ref, gn_ref):
    agg = aa_ref[...] + ab_ref[...]
    hn = jnp.maximum(
        jnp.dot(h_ref[...], w2t_ref[...], preferred_element_type=jnp.float32)
        + jnp.dot(agg, w2b_ref[...], preferred_element_type=jnp.float32), 0.0)
    hn_ref[...] = hn
    gn_ref[...] = jnp.dot(hn, w1a_ref[...], preferred_element_type=jnp.float32)


_hup_kernel = pl.pallas_call(
    _hup_body,
    grid=(N_NODES // _BN,),
    in_specs=[
        pl.BlockSpec((_BN, D_FEAT), lambda i: (i, 0)),
        pl.BlockSpec((_BN, D_FEAT), lambda i: (i, 0)),
        pl.BlockSpec((_BN, D_FEAT), lambda i: (i, 0)),
        pl.BlockSpec((D_FEAT, D_FEAT), lambda i: (0, 0)),
        pl.BlockSpec((D_FEAT, D_FEAT), lambda i: (0, 0)),
        pl.BlockSpec((D_FEAT, D_FEAT), lambda i: (0, 0)),
    ],
    out_specs=[
        pl.BlockSpec((_BN, D_FEAT), lambda i: (i, 0)),
        pl.BlockSpec((_BN, D_FEAT), lambda i: (i, 0)),
    ],
    out_shape=[
        jax.ShapeDtypeStruct((N_NODES, D_FEAT), jnp.float32),
        jax.ShapeDtypeStruct((N_NODES, D_FEAT), jnp.float32),
    ],
)


def _dec_body(h_ref, aa_ref, ab_ref, w2t_ref, w2b_ref,
              fc1w_ref, fc1b_ref, fc2w_ref, fc2b_ref, out_ref):
    agg = aa_ref[...] + ab_ref[...]
    h2 = jnp.maximum(
        jnp.dot(h_ref[...], w2t_ref[...], preferred_element_type=jnp.float32)
        + jnp.dot(agg, w2b_ref[...], preferred_element_type=jnp.float32), 0.0)
    emb = jnp.dot(h2, fc1w_ref[...],
                  preferred_element_type=jnp.float32) + fc1b_ref[...]
    r = jnp.maximum(emb, 0.0)
    o = jnp.dot(r, fc2w_ref[...],
                preferred_element_type=jnp.float32) + fc2b_ref[...]
    soft = jnp.exp(o - jnp.max(o))
    out_ref[...] = soft / jnp.sum(soft)


_dec_kernel = pl.pallas_call(
    _dec_body,
    out_shape=jax.ShapeDtypeStruct((N_NODES, D_OUT), jnp.float32),
)


def kernel(x, t, edge_index, edge_attr, time_w, time_b,
           W1_0, W2_0, W1_1, W2_1, fc1_w, fc1_b, fc2_w, fc2_b):
    src = edge_index[0].astype(jnp.int32)
    dst = edge_index[1].astype(jnp.int32)

    dt = _dt_kernel(t, src, dst)

    ec0, ec1 = _ec_kernel(
        dt.reshape(N_EDGES, 1), edge_attr,
        time_w.reshape(1, D_FEAT), time_b.reshape(1, D_FEAT),
        W1_0[D_FEAT:D_FEAT + D_EDGE], W1_0[D_FEAT + D_EDGE:],
        W1_1[D_FEAT:D_FEAT + D_EDGE], W1_1[D_FEAT + D_EDGE:])

    g0 = _g_kernel(x, W1_0[:D_FEAT])

    p0 = _edge_kernel(g0, ec0, src, dst)
    h1, g1 = _hup_kernel(x, p0[0], p0[1],
                         W2_0[:D_FEAT], W2_0[D_FEAT:], W1_1[:D_FEAT])

    p1 = _edge_kernel(g1, ec1, src, dst)
    out = _dec_kernel(h1, p1[0], p1[1], W2_1[:D_FEAT], W2_1[D_FEAT:],
                      fc1_w, fc1_b.reshape(1, D_HID),
                      fc2_w, fc2_b.reshape(1, D_OUT))
    return out


# speedup vs baseline: 12.3823x; 1.8747x over previous
import functools

import jax
import jax.numpy as jnp
from jax import lax
from jax.experimental import pallas as pl
from jax.experimental.pallas import tpu as pltpu
from jax.experimental.pallas import tpu_sc as plsc

N_NODES = 10000
N_EDGES = 320000
D_FEAT = 128
D_EDGE = 20
D_HID = 256
D_OUT = 2

NC = 2
NS = 16
NW = NC * NS
EPW = N_EDGES // NW
CHUNK = 40
NCHUNK = EPW // CHUNK
TAIL = EPW - NCHUNK * CHUNK
LANES = 16
STRIPE = 640
WCHUNK = 40

_SC_MESH = plsc.VectorSubcoreMesh(core_axis_name="c", subcore_axis_name="s")
_SC_PARAMS = pltpu.CompilerParams(needs_layout_passes=False)


def _dt_body(t_hbm, src_hbm, dst_hbm, dt_hbm, t_v, src_v, dst_v, dt_v):
    c = lax.axis_index("c")
    s = lax.axis_index("s")
    wid = s * NC + c
    base = wid * EPW
    pltpu.sync_copy(t_hbm, t_v)
    pltpu.sync_copy(src_hbm.at[pl.ds(base, EPW)], src_v)
    pltpu.sync_copy(dst_hbm.at[pl.ds(base, EPW)], dst_v)

    def body(i, carry):
        sl = pl.ds(i * LANES, LANES)
        ts = plsc.load_gather(t_v, [src_v[sl]])
        td = plsc.load_gather(t_v, [dst_v[sl]])
        dt_v[sl] = td - ts
        return carry

    lax.fori_loop(0, EPW // LANES, body, 0)
    pltpu.sync_copy(dt_v, dt_hbm.at[pl.ds(base, EPW)])


_dt_kernel = pl.kernel(
    _dt_body,
    mesh=_SC_MESH,
    compiler_params=_SC_PARAMS,
    out_type=jax.ShapeDtypeStruct((N_EDGES,), jnp.float32),
    scratch_types=[
        pltpu.VMEM((N_NODES,), jnp.float32),
        pltpu.VMEM((EPW,), jnp.int32),
        pltpu.VMEM((EPW,), jnp.int32),
        pltpu.VMEM((EPW,), jnp.float32),
    ],
)


_BE = 3200


_BR = _BE // D_FEAT

_INV2PI = 0.15915493667125702
_C1 = 6.283185482025146
_C2 = -1.7484555314695172e-07
_COS_COEF = (1.0, -0.5, 0.0416666679084301, -0.0013888889225199819,
             2.4801562176435255e-05, -2.755658954356477e-07)


def _fast_cos(x):
    q = jnp.round(x * _INV2PI)
    r = (x - q * _C1) - q * _C2
    r2 = r * r
    acc = jnp.full_like(r2, _COS_COEF[-1])
    for coef in _COS_COEF[-2::-1]:
        acc = acc * r2 + coef
    return acc


def _ec_body(dt_ref, eat_ref, w_ref, b_ref, wb0_ref, wc0_ref, wb1_ref,
             wc1_ref, ec0_ref, ec1_ref):
    contract0 = (((0,), (0,)), ((), ()))
    eat = eat_ref[...]
    ea0 = lax.dot_general(eat, wb0_ref[...], contract0,
                          preferred_element_type=jnp.float32)
    ea1 = lax.dot_general(eat, wb1_ref[...], contract0,
                          preferred_element_type=jnp.float32)
    w = w_ref[...]
    b = b_ref[...]
    wc0 = wc0_ref[...]
    wc1 = wc1_ref[...]
    i = pl.program_id(0)
    for g in range(_BR):
        dtg = dt_ref[i * _BR + g]
        tet = _fast_cos(w * dtg[None, :] + b).astype(jnp.bfloat16)
        rows = pl.ds(g * D_FEAT, D_FEAT)
        ec0_ref[rows, :] = (
            ea0[g * D_FEAT:(g + 1) * D_FEAT, :]
            + lax.dot_general(tet, wc0, contract0,
                              preferred_element_type=jnp.float32))
        ec1_ref[rows, :] = (
            ea1[g * D_FEAT:(g + 1) * D_FEAT, :]
            + lax.dot_general(tet, wc1, contract0,
                              preferred_element_type=jnp.float32))


_ec_kernel = pl.pallas_call(
    _ec_body,
    grid=(N_EDGES // _BE,),
    in_specs=[
        pl.BlockSpec((N_EDGES // D_FEAT, D_FEAT), lambda i: (0, 0)),
        pl.BlockSpec((D_EDGE, _BE), lambda i: (0, i)),
        pl.BlockSpec((D_FEAT, 1), lambda i: (0, 0)),
        pl.BlockSpec((D_FEAT, 1), lambda i: (0, 0)),
        pl.BlockSpec((D_EDGE, D_FEAT), lambda i: (0, 0)),
        pl.BlockSpec((D_FEAT, D_FEAT), lambda i: (0, 0)),
        pl.BlockSpec((D_EDGE, D_FEAT), lambda i: (0, 0)),
        pl.BlockSpec((D_FEAT, D_FEAT), lambda i: (0, 0)),
    ],
    out_specs=[
        pl.BlockSpec((_BE, D_FEAT), lambda i: (i, 0)),
        pl.BlockSpec((_BE, D_FEAT), lambda i: (i, 0)),
    ],
    out_shape=[
        jax.ShapeDtypeStruct((N_EDGES, D_FEAT), jnp.float32),
        jax.ShapeDtypeStruct((N_EDGES, D_FEAT), jnp.float32),
    ],
)


_BN = 1000


def _g_body(h_ref, w_ref, g_ref):
    g_ref[...] = jnp.dot(h_ref[...], w_ref[...],
                         preferred_element_type=jnp.float32)


_g_kernel = pl.pallas_call(
    _g_body,
    grid=(N_NODES // _BN,),
    in_specs=[
        pl.BlockSpec((_BN, D_FEAT), lambda i: (i, 0)),
        pl.BlockSpec((D_FEAT, D_FEAT), lambda i: (0, 0)),
    ],
    out_specs=pl.BlockSpec((_BN, D_FEAT), lambda i: (i, 0)),
    out_shape=jax.ShapeDtypeStruct((N_NODES, D_FEAT), jnp.float32),
)


def _edge_body(g_hbm, ec_hbm, src_hbm, dst_hbm, out_hbm,
               src_all, dst_c2, gb, eb, ob,
               agg_sh, gsems, esems, dsems, ssems):
    c = lax.axis_index("c")
    s = lax.axis_index("s")
    wid = s * NC + c
    base = wid * EPW
    nwb = jnp.where(s < NS - 1, STRIPE // WCHUNK, (N_NODES - (NS - 1) * STRIPE) // WCHUNK)

    def zrow(i, carry):
        for j in range(D_FEAT // LANES):
            ob[0, i, pl.ds(j * LANES, LANES)] = jnp.zeros((LANES,), jnp.float32)
        return carry

    lax.fori_loop(0, WCHUNK, zrow, 0)

    def zstripe(q, carry):
        pltpu.sync_copy(ob.at[0, pl.ds(0, WCHUNK)],
                        agg_sh.at[pl.ds(s * STRIPE + q * WCHUNK, WCHUNK)])
        return carry

    lax.fori_loop(0, nwb, zstripe, 0)
    pltpu.sync_copy(src_hbm.at[pl.ds(base, EPW)], src_all)
    plsc.subcore_barrier()

    def prefetch(ci, b, d):
        e0 = base + ci * CHUNK
        pltpu.async_copy(dst_hbm.at[pl.ds(e0, CHUNK)], dst_c2.at[d], dsems[d])
        pltpu.async_copy(g_hbm.at[src_all.at[pl.ds(ci * CHUNK, CHUNK)]],
                         gb.at[b], gsems[b])
        pltpu.async_copy(ec_hbm.at[pl.ds(e0, CHUNK)], eb.at[b], esems[b])

    def process(ci, b, d, maybe_first, do_prefetch):
        pltpu.make_async_copy(g_hbm.at[src_all.at[pl.ds(0, CHUNK)]],
                              gb.at[b], gsems[b]).wait()
        pltpu.make_async_copy(ec_hbm.at[pl.ds(base, CHUNK)],
                              eb.at[b], esems[b]).wait()

        def swait():
            pltpu.make_async_copy(ob.at[b], agg_sh.at[dst_c2.at[d]],
                                  ssems[b]).wait()

        if maybe_first is None:
            swait()
        else:
            pl.when(maybe_first > 0)(swait)

        def row(i, rcarry):
            for j in range(D_FEAT // LANES):
                sl = pl.ds(j * LANES, LANES)
                ob[b, i, sl] = jnp.maximum(gb[b, i, sl] + eb[b, i, sl], 0.0)
            return rcarry

        lax.fori_loop(0, CHUNK, row, 0)
        pltpu.make_async_copy(dst_hbm.at[pl.ds(base, CHUNK)],
                              dst_c2.at[d], dsems[d]).wait()
        pltpu.async_copy(ob.at[b], agg_sh.at[dst_c2.at[d]], ssems[b], add=True)
        if do_prefetch:
            prefetch(ci + 2, b, (d + 2) % 4)

    prefetch(0, 0, 0)
    prefetch(1, 1, 1)

    NQ = (NCHUNK - 2) // 4

    def quad_body(q, carry):
        for u in range(4):
            process(4 * q + u, u & 1, u, q if u < 2 else None, True)
        return carry

    lax.fori_loop(0, NQ, quad_body, 0)
    process(NCHUNK - 2, 0, (NCHUNK - 2) % 4, None, False)
    process(NCHUNK - 1, 1, (NCHUNK - 1) % 4, None, False)

    for b in range(2):
        pltpu.make_async_copy(ob.at[b], agg_sh.at[dst_c2.at[b]],
                              ssems[b]).wait()
    plsc.subcore_barrier()

    def wstripe(q, carry):
        r0 = s * STRIPE + q * WCHUNK
        pltpu.sync_copy(agg_sh.at[pl.ds(r0, WCHUNK)], ob.at[0, pl.ds(0, WCHUNK)])
        pltpu.sync_copy(ob.at[0, pl.ds(0, WCHUNK)], out_hbm.at[c, pl.ds(r0, WCHUNK)])
        return carry

    lax.fori_loop(0, nwb, wstripe, 0)


_edge_kernel = pl.kernel(
    _edge_body,
    mesh=_SC_MESH,
    compiler_params=_SC_PARAMS,
    out_type=jax.ShapeDtypeStruct((NC, N_NODES, D_FEAT), jnp.float32),
    scratch_types=[
        pltpu.VMEM((EPW,), jnp.int32),
        pltpu.VMEM((4, CHUNK), jnp.int32),
        pltpu.VMEM((2, CHUNK, D_FEAT), jnp.float32),
        pltpu.VMEM((2, CHUNK, D_FEAT), jnp.float32),
        pltpu.VMEM((2, CHUNK, D_FEAT), jnp.float32),
        pltpu.VMEM_SHARED((N_NODES, D_FEAT), jnp.float32),
        [pltpu.SemaphoreType.DMA, pltpu.SemaphoreType.DMA],
        [pltpu.SemaphoreType.DMA, pltpu.SemaphoreType.DMA],
        [pltpu.SemaphoreType.DMA, pltpu.SemaphoreType.DMA,
         pltpu.SemaphoreType.DMA, pltpu.SemaphoreType.DMA],
        [pltpu.SemaphoreType.DMA, pltpu.SemaphoreType.DMA],
    ],
)


def _hup_body(h_ref, aa_ref, ab_ref, w2t_ref, w2b_ref, w1a_ref, hn_ref, gn_ref):
    agg = aa_ref[...] + ab_ref[...]
    hn = jnp.maximum(
        jnp.dot(h_ref[...], w2t_ref[...], preferred_element_type=jnp.float32)
        + jnp.dot(agg, w2b_ref[...], preferred_element_type=jnp.float32), 0.0)
    hn_ref[...] = hn
    gn_ref[...] = jnp.dot(hn, w1a_ref[...], preferred_element_type=jnp.float32)


_hup_kernel = pl.pallas_call(
    _hup_body,
    grid=(N_NODES // _BN,),
    in_specs=[
        pl.BlockSpec((_BN, D_FEAT), lambda i: (i, 0)),
        pl.BlockSpec((_BN, D_FEAT), lambda i: (i, 0)),
        pl.BlockSpec((_BN, D_FEAT), lambda i: (i, 0)),
        pl.BlockSpec((D_FEAT, D_FEAT), lambda i: (0, 0)),
        pl.BlockSpec((D_FEAT, D_FEAT), lambda i: (0, 0)),
        pl.BlockSpec((D_FEAT, D_FEAT), lambda i: (0, 0)),
    ],
    out_specs=[
        pl.BlockSpec((_BN, D_FEAT), lambda i: (i, 0)),
        pl.BlockSpec((_BN, D_FEAT), lambda i: (i, 0)),
    ],
    out_shape=[
        jax.ShapeDtypeStruct((N_NODES, D_FEAT), jnp.float32),
        jax.ShapeDtypeStruct((N_NODES, D_FEAT), jnp.float32),
    ],
)


def _dec_body(h_ref, aa_ref, ab_ref, w2t_ref, w2b_ref,
              fc1w_ref, fc1b_ref, fc2w_ref, fc2b_ref, out_ref):
    agg = aa_ref[...] + ab_ref[...]
    h2 = jnp.maximum(
        jnp.dot(h_ref[...], w2t_ref[...], preferred_element_type=jnp.float32)
        + jnp.dot(agg, w2b_ref[...], preferred_element_type=jnp.float32), 0.0)
    emb = jnp.dot(h2, fc1w_ref[...],
                  preferred_element_type=jnp.float32) + fc1b_ref[...]
    r = jnp.maximum(emb, 0.0)
    o = jnp.dot(r, fc2w_ref[...],
                preferred_element_type=jnp.float32) + fc2b_ref[...]
    soft = jnp.exp(o - jnp.max(o))
    out_ref[...] = soft / jnp.sum(soft)


_dec_kernel = pl.pallas_call(
    _dec_body,
    out_shape=jax.ShapeDtypeStruct((N_NODES, D_OUT), jnp.float32),
)


def kernel(x, t, edge_index, edge_attr, time_w, time_b,
           W1_0, W2_0, W1_1, W2_1, fc1_w, fc1_b, fc2_w, fc2_b):
    src = edge_index[0].astype(jnp.int32)
    dst = edge_index[1].astype(jnp.int32)

    dt = _dt_kernel(t, src, dst)

    eat = edge_attr.T.astype(jnp.bfloat16)
    w2 = time_w.reshape(D_FEAT, 1)
    b2 = time_b.reshape(D_FEAT, 1)
    dt2 = dt.reshape(N_EDGES // D_FEAT, D_FEAT)
    ec0, ec1 = _ec_kernel(
        dt2, eat, w2, b2,
        W1_0[D_FEAT:D_FEAT + D_EDGE].astype(jnp.bfloat16),
        W1_0[D_FEAT + D_EDGE:].astype(jnp.bfloat16),
        W1_1[D_FEAT:D_FEAT + D_EDGE].astype(jnp.bfloat16),
        W1_1[D_FEAT + D_EDGE:].astype(jnp.bfloat16))

    g0 = _g_kernel(x, W1_0[:D_FEAT])

    p0 = _edge_kernel(g0, ec0, src, dst)
    h1, g1 = _hup_kernel(x, p0[0], p0[1],
                         W2_0[:D_FEAT], W2_0[D_FEAT:], W1_1[:D_FEAT])

    p1 = _edge_kernel(g1, ec1, src, dst)
    out = _dec_kernel(h1, p1[0], p1[1], W2_1[:D_FEAT], W2_1[D_FEAT:],
                      fc1_w, fc1_b.reshape(1, D_HID),
                      fc2_w, fc2_b.reshape(1, D_OUT))
    return out


# speedup vs baseline: 12.9795x; 1.0482x over previous
import functools

import jax
import jax.numpy as jnp
from jax import lax
from jax.experimental import pallas as pl
from jax.experimental.pallas import tpu as pltpu
from jax.experimental.pallas import tpu_sc as plsc

N_NODES = 10000
N_EDGES = 320000
D_FEAT = 128
D_EDGE = 20
D_HID = 256
D_OUT = 2

NC = 2
NS = 16
NW = NC * NS
EPW = N_EDGES // NW
CHUNK = 40
NCHUNK = EPW // CHUNK
TAIL = EPW - NCHUNK * CHUNK
LANES = 16
STRIPE = 640
WCHUNK = 40

_SC_MESH = plsc.VectorSubcoreMesh(core_axis_name="c", subcore_axis_name="s")
_SC_PARAMS = pltpu.CompilerParams(needs_layout_passes=False)


def _dt_body(t_hbm, src_hbm, dst_hbm, dt_hbm, t_v, src_v, dst_v, dt_v):
    c = lax.axis_index("c")
    s = lax.axis_index("s")
    wid = s * NC + c
    base = wid * EPW
    pltpu.sync_copy(t_hbm, t_v)
    pltpu.sync_copy(src_hbm.at[pl.ds(base, EPW)], src_v)
    pltpu.sync_copy(dst_hbm.at[pl.ds(base, EPW)], dst_v)

    def body(i, carry):
        sl = pl.ds(i * LANES, LANES)
        ts = plsc.load_gather(t_v, [src_v[sl]])
        td = plsc.load_gather(t_v, [dst_v[sl]])
        dt_v[sl] = td - ts
        return carry

    lax.fori_loop(0, EPW // LANES, body, 0)
    pltpu.sync_copy(dt_v, dt_hbm.at[pl.ds(base, EPW)])


_dt_kernel = pl.kernel(
    _dt_body,
    mesh=_SC_MESH,
    compiler_params=_SC_PARAMS,
    out_type=jax.ShapeDtypeStruct((N_EDGES,), jnp.float32),
    scratch_types=[
        pltpu.VMEM((N_NODES,), jnp.float32),
        pltpu.VMEM((EPW,), jnp.int32),
        pltpu.VMEM((EPW,), jnp.int32),
        pltpu.VMEM((EPW,), jnp.float32),
    ],
)


_BE = 3200


_BR = _BE // D_FEAT

_INV2PI = 0.15915493667125702
_C1 = 6.283185482025146
_C2 = -1.7484555314695172e-07
_COS_COEF = (1.0, -0.5, 0.0416666679084301, -0.0013888889225199819,
             2.4801562176435255e-05, -2.755658954356477e-07)


def _fast_cos(x):
    q = jnp.round(x * _INV2PI)
    r = (x - q * _C1) - q * _C2
    r2 = r * r
    acc = jnp.full_like(r2, _COS_COEF[-1])
    for coef in _COS_COEF[-2::-1]:
        acc = acc * r2 + coef
    return acc


def _ec_body(dt_ref, eat_ref, w_ref, b_ref, wb_ref, wc_ref, ec_ref):
    contract0 = (((0,), (0,)), ((), ()))
    ea = lax.dot_general(eat_ref[...], wb_ref[...], contract0,
                         preferred_element_type=jnp.float32)
    w = w_ref[...]
    b = b_ref[...]
    wc = wc_ref[...]
    i = pl.program_id(0)
    for g in range(_BR):
        dtg = dt_ref[i * _BR + g]
        tet = _fast_cos(w * dtg[None, :] + b).astype(jnp.bfloat16)
        ec_ref[pl.ds(g * D_FEAT, D_FEAT), :] = (
            ea[g * D_FEAT:(g + 1) * D_FEAT, :]
            + lax.dot_general(tet, wc, contract0,
                              preferred_element_type=jnp.float32))


_ec_kernel = pl.pallas_call(
    _ec_body,
    grid=(N_EDGES // _BE,),
    in_specs=[
        pl.BlockSpec((N_EDGES // D_FEAT, D_FEAT), lambda i: (0, 0)),
        pl.BlockSpec((D_EDGE, _BE), lambda i: (0, i)),
        pl.BlockSpec((D_FEAT, 1), lambda i: (0, 0)),
        pl.BlockSpec((D_FEAT, 1), lambda i: (0, 0)),
        pl.BlockSpec((D_EDGE, D_FEAT), lambda i: (0, 0)),
        pl.BlockSpec((D_FEAT, D_FEAT), lambda i: (0, 0)),
    ],
    out_specs=pl.BlockSpec((_BE, D_FEAT), lambda i: (i, 0)),
    out_shape=jax.ShapeDtypeStruct((N_EDGES, D_FEAT), jnp.float32),
)


_BN = 1000


def _g_body(h_ref, w_ref, g_ref):
    g_ref[...] = jnp.dot(h_ref[...], w_ref[...],
                         preferred_element_type=jnp.float32)


_g_kernel = pl.pallas_call(
    _g_body,
    grid=(N_NODES // _BN,),
    in_specs=[
        pl.BlockSpec((_BN, D_FEAT), lambda i: (i, 0)),
        pl.BlockSpec((D_FEAT, D_FEAT), lambda i: (0, 0)),
    ],
    out_specs=pl.BlockSpec((_BN, D_FEAT), lambda i: (i, 0)),
    out_shape=jax.ShapeDtypeStruct((N_NODES, D_FEAT), jnp.float32),
)


def _edge_body(g_hbm, ec_hbm, src_hbm, dst_hbm, out_hbm,
               src_all, dst_c2, gb, eb, ob,
               agg_sh, gsems, esems, dsems, ssems):
    c = lax.axis_index("c")
    s = lax.axis_index("s")
    wid = s * NC + c
    base = wid * EPW
    nwb = jnp.where(s < NS - 1, STRIPE // WCHUNK, (N_NODES - (NS - 1) * STRIPE) // WCHUNK)

    def zrow(i, carry):
        for j in range(D_FEAT // LANES):
            ob[0, i, pl.ds(j * LANES, LANES)] = jnp.zeros((LANES,), jnp.float32)
        return carry

    lax.fori_loop(0, WCHUNK, zrow, 0)

    def zstripe(q, carry):
        pltpu.sync_copy(ob.at[0, pl.ds(0, WCHUNK)],
                        agg_sh.at[pl.ds(s * STRIPE + q * WCHUNK, WCHUNK)])
        return carry

    lax.fori_loop(0, nwb, zstripe, 0)
    pltpu.sync_copy(src_hbm.at[pl.ds(base, EPW)], src_all)
    plsc.subcore_barrier()

    def prefetch(ci, b, d):
        e0 = base + ci * CHUNK
        pltpu.async_copy(dst_hbm.at[pl.ds(e0, CHUNK)], dst_c2.at[d], dsems[d])
        pltpu.async_copy(g_hbm.at[src_all.at[pl.ds(ci * CHUNK, CHUNK)]],
                         gb.at[b], gsems[b])
        pltpu.async_copy(ec_hbm.at[pl.ds(e0, CHUNK)], eb.at[b], esems[b])

    def process(ci, b, d, maybe_first, do_prefetch):
        pltpu.make_async_copy(g_hbm.at[src_all.at[pl.ds(0, CHUNK)]],
                              gb.at[b], gsems[b]).wait()
        pltpu.make_async_copy(ec_hbm.at[pl.ds(base, CHUNK)],
                              eb.at[b], esems[b]).wait()

        def swait():
            pltpu.make_async_copy(ob.at[b], agg_sh.at[dst_c2.at[d]],
                                  ssems[b]).wait()

        if maybe_first is None:
            swait()
        else:
            pl.when(maybe_first > 0)(swait)

        def row(i, rcarry):
            for j in range(D_FEAT // LANES):
                sl = pl.ds(j * LANES, LANES)
                ob[b, i, sl] = jnp.maximum(gb[b, i, sl] + eb[b, i, sl], 0.0)
            return rcarry

        lax.fori_loop(0, CHUNK, row, 0)
        pltpu.make_async_copy(dst_hbm.at[pl.ds(base, CHUNK)],
                              dst_c2.at[d], dsems[d]).wait()
        pltpu.async_copy(ob.at[b], agg_sh.at[dst_c2.at[d]], ssems[b], add=True)
        if do_prefetch:
            prefetch(ci + 2, b, (d + 2) % 4)

    prefetch(0, 0, 0)
    prefetch(1, 1, 1)

    NQ = (NCHUNK - 2) // 4

    def quad_body(q, carry):
        for u in range(4):
            process(4 * q + u, u & 1, u, q if u < 2 else None, True)
        return carry

    lax.fori_loop(0, NQ, quad_body, 0)
    process(NCHUNK - 2, 0, (NCHUNK - 2) % 4, None, False)
    process(NCHUNK - 1, 1, (NCHUNK - 1) % 4, None, False)

    for b in range(2):
        pltpu.make_async_copy(ob.at[b], agg_sh.at[dst_c2.at[b]],
                              ssems[b]).wait()
    plsc.subcore_barrier()

    def wstripe(q, carry):
        r0 = s * STRIPE + q * WCHUNK
        pltpu.sync_copy(agg_sh.at[pl.ds(r0, WCHUNK)], ob.at[0, pl.ds(0, WCHUNK)])
        pltpu.sync_copy(ob.at[0, pl.ds(0, WCHUNK)], out_hbm.at[c, pl.ds(r0, WCHUNK)])
        return carry

    lax.fori_loop(0, nwb, wstripe, 0)


_edge_kernel = pl.kernel(
    _edge_body,
    mesh=_SC_MESH,
    compiler_params=_SC_PARAMS,
    out_type=jax.ShapeDtypeStruct((NC, N_NODES, D_FEAT), jnp.float32),
    scratch_types=[
        pltpu.VMEM((EPW,), jnp.int32),
        pltpu.VMEM((4, CHUNK), jnp.int32),
        pltpu.VMEM((2, CHUNK, D_FEAT), jnp.float32),
        pltpu.VMEM((2, CHUNK, D_FEAT), jnp.float32),
        pltpu.VMEM((2, CHUNK, D_FEAT), jnp.float32),
        pltpu.VMEM_SHARED((N_NODES, D_FEAT), jnp.float32),
        [pltpu.SemaphoreType.DMA, pltpu.SemaphoreType.DMA],
        [pltpu.SemaphoreType.DMA, pltpu.SemaphoreType.DMA],
        [pltpu.SemaphoreType.DMA, pltpu.SemaphoreType.DMA,
         pltpu.SemaphoreType.DMA, pltpu.SemaphoreType.DMA],
        [pltpu.SemaphoreType.DMA, pltpu.SemaphoreType.DMA],
    ],
)


def _hup_body(h_ref, aa_ref, ab_ref, w2t_ref, w2b_ref, w1a_ref, hn_ref, gn_ref):
    agg = aa_ref[0] + ab_ref[0]
    hn = jnp.maximum(
        jnp.dot(h_ref[...], w2t_ref[...], preferred_element_type=jnp.float32)
        + jnp.dot(agg, w2b_ref[...], preferred_element_type=jnp.float32), 0.0)
    hn_ref[...] = hn
    gn_ref[...] = jnp.dot(hn, w1a_ref[...], preferred_element_type=jnp.float32)


_hup_kernel = pl.pallas_call(
    _hup_body,
    grid=(N_NODES // _BN,),
    in_specs=[
        pl.BlockSpec((_BN, D_FEAT), lambda i: (i, 0)),
        pl.BlockSpec((1, _BN, D_FEAT), lambda i: (0, i, 0)),
        pl.BlockSpec((1, _BN, D_FEAT), lambda i: (1, i, 0)),
        pl.BlockSpec((D_FEAT, D_FEAT), lambda i: (0, 0)),
        pl.BlockSpec((D_FEAT, D_FEAT), lambda i: (0, 0)),
        pl.BlockSpec((D_FEAT, D_FEAT), lambda i: (0, 0)),
    ],
    out_specs=[
        pl.BlockSpec((_BN, D_FEAT), lambda i: (i, 0)),
        pl.BlockSpec((_BN, D_FEAT), lambda i: (i, 0)),
    ],
    out_shape=[
        jax.ShapeDtypeStruct((N_NODES, D_FEAT), jnp.float32),
        jax.ShapeDtypeStruct((N_NODES, D_FEAT), jnp.float32),
    ],
)


def _dec_body(h_ref, p_ref, w2t_ref, w2b_ref,
              fc1w_ref, fc1b_ref, fc2w_ref, fc2b_ref, out_ref):
    agg = p_ref[0] + p_ref[1]
    h2 = jnp.maximum(
        jnp.dot(h_ref[...], w2t_ref[...], preferred_element_type=jnp.float32)
        + jnp.dot(agg, w2b_ref[...], preferred_element_type=jnp.float32), 0.0)
    emb = jnp.dot(h2, fc1w_ref[...],
                  preferred_element_type=jnp.float32) + fc1b_ref[...]
    r = jnp.maximum(emb, 0.0)
    o = jnp.dot(r, fc2w_ref[...],
                preferred_element_type=jnp.float32) + fc2b_ref[...]
    soft = jnp.exp(o - jnp.max(o))
    out_ref[...] = soft / jnp.sum(soft)


_dec_kernel = pl.pallas_call(
    _dec_body,
    out_shape=jax.ShapeDtypeStruct((N_NODES, D_OUT), jnp.float32),
)


def kernel(x, t, edge_index, edge_attr, time_w, time_b,
           W1_0, W2_0, W1_1, W2_1, fc1_w, fc1_b, fc2_w, fc2_b):
    src = edge_index[0].astype(jnp.int32)
    dst = edge_index[1].astype(jnp.int32)

    dt = _dt_kernel(t, src, dst)

    eat = edge_attr.T.astype(jnp.bfloat16)
    w2 = time_w.reshape(D_FEAT, 1)
    b2 = time_b.reshape(D_FEAT, 1)
    dt2 = dt.reshape(N_EDGES // D_FEAT, D_FEAT)
    ec0 = _ec_kernel(dt2, eat, w2, b2,
                     W1_0[D_FEAT:D_FEAT + D_EDGE].astype(jnp.bfloat16),
                     W1_0[D_FEAT + D_EDGE:].astype(jnp.bfloat16))
    ec1 = _ec_kernel(dt2, eat, w2, b2,
                     W1_1[D_FEAT:D_FEAT + D_EDGE].astype(jnp.bfloat16),
                     W1_1[D_FEAT + D_EDGE:].astype(jnp.bfloat16))

    g0 = _g_kernel(x, W1_0[:D_FEAT])

    p0 = _edge_kernel(g0, ec0, src, dst)
    h1, g1 = _hup_kernel(x, p0, p0,
                         W2_0[:D_FEAT], W2_0[D_FEAT:], W1_1[:D_FEAT])

    p1 = _edge_kernel(g1, ec1, src, dst)
    out = _dec_kernel(h1, p1, W2_1[:D_FEAT], W2_1[D_FEAT:],
                      fc1_w, fc1_b.reshape(1, D_HID),
                      fc2_w, fc2_b.reshape(1, D_OUT))
    return out
